# baseline (speedup 1.0000x reference)
import numpy as np

import jax
import jax.numpy as jnp
from jax import lax
from jax.experimental import pallas as pl
from jax.experimental.pallas import tpu as pltpu
from jax.experimental.pallas import tpu_sc as plsc

N = 10000
E = 160000
NB = 2000
B = 4
HID = 256
EMB = 128
EDG = 64
NAT = 30
NBT = 50

NP = 10240
EP = 163840
NBP = 2048
NC = 2
NS = 16
KE = 80
HALF = 128

_mesh = plsc.VectorSubcoreMesh(core_axis_name="c", subcore_axis_name="s")
_SC_PARAMS = pltpu.CompilerParams(use_tc_tiling_on_sc=False,
                                  needs_layout_passes=False)


def _tables_body(atom_embed, W_e2h, block_embed, s_t2, batch2, t2, b_e2h,
                 b_tp, W_tp, edge_embed, Wme1, bm1, Wme2, bm2,
                 ae2_o, bet_o, et1_o, et2_o):
    ae2_o[...] = jnp.dot(atom_embed[...], W_e2h[...],
                         preferred_element_type=jnp.float32)
    t = t2[...][:, :1]
    half = HID // 2
    ii = lax.broadcasted_iota(jnp.int32, (1, half), 1).astype(jnp.float32)
    freqs = jnp.exp(-np.log(10000.0) * ii / (half - 1))
    ang = t * freqs
    te = jnp.concatenate([jnp.sin(ang), jnp.cos(ang)], axis=-1)
    te4 = jnp.dot(te, W_tp[...], preferred_element_type=jnp.float32)
    sids = s_t2[...]
    oh_s = (sids == lax.broadcasted_iota(jnp.int32, (NBP, NBT), 1)
            ).astype(jnp.float32)
    be = jnp.dot(oh_s, block_embed[...], preferred_element_type=jnp.float32)
    be2 = jnp.dot(be, W_e2h[...], preferred_element_type=jnp.float32)
    bids = batch2[...]
    oh_b = (bids == lax.broadcasted_iota(jnp.int32, (NBP, 16), 1)
            ).astype(jnp.float32)
    tpart = jnp.dot(oh_b, te4, preferred_element_type=jnp.float32)
    bet_o[...] = be2 + tpart + b_e2h[...][0] + b_tp[...][0]
    e1 = jnp.dot(edge_embed[...], Wme1[...],
                 preferred_element_type=jnp.float32) + bm1[...][0]
    e2 = jnp.dot(edge_embed[...], Wme2[...],
                 preferred_element_type=jnp.float32) + bm2[...][0]
    zpad = jnp.zeros((5, HID), jnp.float32)
    et1_o[...] = jnp.concatenate([e1, zpad], axis=0)
    et2_o[...] = jnp.concatenate([e2, zpad], axis=0)


def _tables(*args):
    return pl.pallas_call(
        _tables_body,
        out_shape=[
            jax.ShapeDtypeStruct((NAT, HID), jnp.float32),
            jax.ShapeDtypeStruct((NBP, HID), jnp.float32),
            jax.ShapeDtypeStruct((8, HID), jnp.float32),
            jax.ShapeDtypeStruct((8, HID), jnp.float32),
        ],
    )(*args)


def _enc_body(ae2_hbm, bet_hbm, at_hbm, bid_hbm, x0_hbm, gm_hbm,
              h0_hbm, xb_hbm, idxa, idxb, bufa, bufb, bufx, bufg):
    cid = lax.axis_index("c")
    sid = lax.axis_index("s")
    wid = sid * NC + cid
    per_w = NP // (NC * NS)
    nchunk = per_w // 64

    @pl.loop(0, nchunk)
    def _chunk(k):
        base = wid * per_w + k * 64
        pltpu.sync_copy(at_hbm.at[pl.ds(base, 64)], idxa)
        pltpu.sync_copy(bid_hbm.at[pl.ds(base, 64)], idxb)
        pltpu.sync_copy(ae2_hbm.at[idxa], bufa)
        pltpu.sync_copy(bet_hbm.at[idxb], bufb)
        pltpu.sync_copy(x0_hbm.at[pl.ds(base, 64)], bufx)
        pltpu.sync_copy(gm_hbm.at[idxb], bufg)

        @pl.loop(0, 64)
        def _row(r):
            for c in range(HID // 16):
                sl = pl.ds(c * 16, 16)
                bufa[r, sl] = bufa[r, sl] + bufb[r, sl]
            bufx[r, :] = bufx[r, :] + bufg[r, :]

        pltpu.sync_copy(bufa, h0_hbm.at[pl.ds(base, 64)])
        pltpu.sync_copy(bufx, xb_hbm.at[pl.ds(base, 64)])


def _enc(ae2, bet, at_p, bid_p, xpad0, gmtab):
    f = pl.kernel(
        _enc_body,
        out_type=[
            jax.ShapeDtypeStruct((NP, HID), jnp.float32),
            jax.ShapeDtypeStruct((NP, 16), jnp.float32),
        ],
        mesh=_mesh,
        compiler_params=_SC_PARAMS,
        scratch_types=[
            pltpu.VMEM((64,), jnp.int32),
            pltpu.VMEM((64,), jnp.int32),
            pltpu.VMEM((64, HID), jnp.float32),
            pltpu.VMEM((64, HID), jnp.float32),
            pltpu.VMEM((64, 16), jnp.float32),
            pltpu.VMEM((64, 16), jnp.float32),
        ],
    )
    return f(ae2, bet, at_p, bid_p, xpad0, gmtab)


def _premul_body(h_ref, wms_ref, wmd_ref, et_ref, hs_o, hd_o):
    hcol = pl.program_id(0)
    hblk = h_ref[...]
    ws = wms_ref[:, pl.ds(hcol * HALF, HALF)]
    wd = wmd_ref[:, pl.ds(hcol * HALF, HALF)]
    hs_o[...] = jnp.dot(hblk, ws, preferred_element_type=jnp.float32
                        )[None].astype(jnp.bfloat16)
    hd = jnp.dot(hblk, wd, preferred_element_type=jnp.float32)
    ets = et_ref[pl.ds(hcol * 8, 8)]
    for tt in range(3):
        hd_o[0, tt] = (hd + ets[tt]).astype(jnp.bfloat16)


def _premul(h, wms, wmd, et16):
    bn = 512
    return pl.pallas_call(
        _premul_body,
        grid=(NC, NP // bn),
        in_specs=[
            pl.BlockSpec((bn, HID), lambda hh, i: (i, 0)),
            pl.BlockSpec((HID, HID), lambda hh, i: (0, 0)),
            pl.BlockSpec((HID, HID), lambda hh, i: (0, 0)),
            pl.BlockSpec((16, HALF), lambda hh, i: (0, 0)),
        ],
        out_specs=[
            pl.BlockSpec((1, bn, HALF), lambda hh, i: (hh, i, 0)),
            pl.BlockSpec((1, 3, bn, HALF), lambda hh, i: (hh, 0, i, 0)),
        ],
        out_shape=[
            jax.ShapeDtypeStruct((NC, NP, HALF), jnp.bfloat16),
            jax.ShapeDtypeStruct((NC, 3, NP, HALF), jnp.bfloat16),
        ],
    )(h, wms, wmd, et16)


def _edge_body(hsf_hbm, hdf_hbm, idx3_hbm, z128_hbm, agg_hbm,
               buf3_0, idxr2_0, idxc2_0, bufs_0, bufd_0, bufm_0, idxs_0,
               buf3_1, idxr2_1, idxc2_1, bufs_1, bufd_1, bufm_1, idxs_1,
               semg0, semg1, sems0, sems1, aggsh):
    cid = lax.axis_index("c")
    sid = lax.axis_index("s")
    coff = cid * NP
    coffd = cid * 3 * NP
    per_s = EP // NS
    nchunk = per_s // KE
    rows_per_s = NP // NS

    pltpu.sync_copy(z128_hbm, aggsh.at[pl.ds(sid * rows_per_s, rows_per_s)])
    plsc.subcore_barrier()

    sets = ((buf3_0, idxr2_0, idxc2_0, bufs_0, bufd_0, bufm_0, idxs_0,
             semg0, sems0),
            (buf3_1, idxr2_1, idxc2_1, bufs_1, bufd_1, bufm_1, idxs_1,
             semg1, sems1))

    def scatter_wait(b):
        _, _, _, _, _, bufm, idxs, _, sems = sets[b]
        pltpu.make_async_copy(bufm, aggsh.at[idxs.at[0]], sems).wait()

    def load_and_gather(k, b):
        buf3, idxr2, idxc2, bufs, bufd, _, _, semg, _ = sets[b]
        base = sid * per_s + k * KE
        pltpu.sync_copy(idx3_hbm.at[:, pl.ds(base, KE)], buf3)
        for j in range(KE // 16):
            sl = pl.ds(j * 16, 16)
            idxr2[sl] = buf3[0, sl] + coff
            idxc2[sl] = buf3[1, sl] + buf3[2, sl] * NP + coffd
        pltpu.async_copy(hsf_hbm.at[idxr2], bufs, semg)
        pltpu.async_copy(hdf_hbm.at[idxc2], bufd, semg)

    def wait_gather(b):
        buf3, idxr2, idxc2, bufs, bufd, _, _, semg, _ = sets[b]
        pltpu.make_async_copy(hsf_hbm.at[idxr2], bufs, semg).wait()
        pltpu.make_async_copy(hdf_hbm.at[idxc2], bufd, semg).wait()

    load_and_gather(0, 0)
    load_and_gather(1, 1)

    @pl.loop(0, nchunk, step=2)
    def _pair(k):
        for b in range(2):
            kk = k + b
            buf3, _, _, bufs, bufd, bufm, idxs, _, sems = sets[b]
            wait_gather(b)

            @pl.when(kk >= 2)
            def _():
                scatter_wait(b)

            @pl.loop(0, KE)
            def _edge(e):
                for c in range(4):
                    sl = pl.ds(c * 32, 32)
                    vm = jnp.maximum(bufs[e, sl] + bufd[e, sl],
                                     jnp.bfloat16(0.0))
                    lo, hi = plsc.unpack(vm,
                                         format=plsc.PackFormat.INTERLEAVED)
                    bufm[e, pl.ds(c * 32, 16)] = lo
                    bufm[e, pl.ds(c * 32 + 16, 16)] = hi

            for j in range(KE // 16):
                sl = pl.ds(j * 16, 16)
                idxs[0, sl] = buf3[0, sl]
            pltpu.async_copy(bufm, aggsh.at[idxs.at[0]], sems, add=True)

            @pl.when(kk + 2 < nchunk)
            def _():
                load_and_gather(kk + 2, b)

    scatter_wait(0)
    scatter_wait(1)
    plsc.subcore_barrier()
    pltpu.sync_copy(aggsh.at[pl.ds(sid * rows_per_s, rows_per_s)],
                    agg_hbm.at[cid, pl.ds(sid * rows_per_s, rows_per_s)])


def _edge_stage(hsf, hdf, idx3, z128):
    bufset = [
        pltpu.VMEM((3, KE), jnp.int32),
        pltpu.VMEM((KE,), jnp.int32),
        pltpu.VMEM((KE,), jnp.int32),
        pltpu.VMEM((KE, HALF), jnp.bfloat16),
        pltpu.VMEM((KE, HALF), jnp.bfloat16),
        pltpu.VMEM((KE, HALF), jnp.float32),
        pltpu.VMEM((1, KE), jnp.int32),
    ]
    f = pl.kernel(
        _edge_body,
        out_type=jax.ShapeDtypeStruct((NC, NP, HALF), jnp.float32),
        mesh=_mesh,
        compiler_params=_SC_PARAMS,
        scratch_types=bufset + bufset + [
            pltpu.SemaphoreType.DMA,
            pltpu.SemaphoreType.DMA,
            pltpu.SemaphoreType.DMA,
            pltpu.SemaphoreType.DMA,
            pltpu.VMEM_SHARED((NP, HALF), jnp.float32),
        ],
    )
    return f(hsf, hdf, idx3, z128)


def _hupd_body(h_ref, a0_ref, a1_ref, wua_ref, wub_ref, bu_ref, o_ref):
    h = h_ref[...]
    z = jnp.dot(h, wua_ref[...], preferred_element_type=jnp.float32)
    z += jnp.dot(a0_ref[0].astype(jnp.float32), wub_ref[...][:HALF, :],
                 preferred_element_type=jnp.float32)
    z += jnp.dot(a1_ref[0].astype(jnp.float32), wub_ref[...][HALF:, :],
                 preferred_element_type=jnp.float32)
    o_ref[...] = h + jnp.maximum(z + bu_ref[...][0], 0.0)


def _hupdate(h, agg, wua, wub, bu):
    bn = 512
    return pl.pallas_call(
        _hupd_body,
        grid=(NP // bn,),
        in_specs=[
            pl.BlockSpec((bn, HID), lambda i: (i, 0)),
            pl.BlockSpec((1, bn, HALF), lambda i: (0, i, 0)),
            pl.BlockSpec((1, bn, HALF), lambda i: (1, i, 0)),
            pl.BlockSpec((HID, HID), lambda i: (0, 0)),
            pl.BlockSpec((HID, HID), lambda i: (0, 0)),
            pl.BlockSpec((1, HID), lambda i: (0, 0)),
        ],
        out_specs=pl.BlockSpec((bn, HID), lambda i: (i, 0)),
        out_shape=jax.ShapeDtypeStruct((NP, HID), jnp.float32),
    )(h, agg, agg, wua, wub, bu)


def _bond_body(xb_hbm, row_hbm, col_hbm, z16_hbm, bacc_hbm,
               idxr, idxc, xr, xc, outr, outc, accsh):
    cid = lax.axis_index("c")
    sid = lax.axis_index("s")
    per_s = (EP // NC) // NS
    nchunk = per_s // KE
    rows_per_s = NP // NS

    pltpu.sync_copy(z16_hbm, accsh.at[pl.ds(sid * rows_per_s, rows_per_s)])
    plsc.subcore_barrier()

    lane = lax.iota(jnp.int32, 16)
    zeros16 = jnp.zeros((16,), jnp.float32)

    @pl.loop(0, nchunk)
    def _chunk(k):
        base = cid * (EP // NC) + sid * per_s + k * KE
        pltpu.sync_copy(row_hbm.at[pl.ds(base, KE)], idxr)
        pltpu.sync_copy(col_hbm.at[pl.ds(base, KE)], idxc)
        pltpu.sync_copy(xb_hbm.at[idxr], xr)
        pltpu.sync_copy(xb_hbm.at[idxc], xc)

        @pl.loop(0, KE)
        def _edge(e):
            a = xr[e, :]
            b = xc[e, :]
            d = jnp.where(lane < 3, a - b, zeros16)
            s = jnp.maximum(jnp.sum(d * d), 1e-30)
            act = jnp.sum(jnp.where(lane == 3, jnp.maximum(a, b), zeros16))
            sv = jnp.full((16,), s, jnp.float32)
            ri = jnp.int32(0x5F3759DF) - (plsc.bitcast(sv, jnp.int32) >> 1)
            r = plsc.bitcast(ri, jnp.float32)
            r = r * (1.5 - 0.5 * sv * r * r)
            r = r * (1.5 - 0.5 * sv * r * r)
            r = r * (1.5 - 0.5 * sv * r * r)
            dist = sv * r + 1e-8
            inv = 1.0 / dist
            contrib = (act * d) * ((dist - 1.5) * inv * inv)
            outr[e, :] = contrib
            outc[e, :] = -contrib

        pltpu.sync_copy(outr, accsh.at[idxr], add=True)
        pltpu.sync_copy(outc, accsh.at[idxc], add=True)

    plsc.subcore_barrier()
    pltpu.sync_copy(accsh.at[pl.ds(sid * rows_per_s, rows_per_s)],
                    bacc_hbm.at[cid, pl.ds(sid * rows_per_s, rows_per_s)])


def _bond_stage(xpadb, row_p, col_p, z16):
    f = pl.kernel(
        _bond_body,
        out_type=jax.ShapeDtypeStruct((NC, NP, 16), jnp.float32),
        mesh=_mesh,
        compiler_params=_SC_PARAMS,
        scratch_types=[
            pltpu.VMEM((KE,), jnp.int32),
            pltpu.VMEM((KE,), jnp.int32),
            pltpu.VMEM((KE, 16), jnp.float32),
            pltpu.VMEM((KE, 16), jnp.float32),
            pltpu.VMEM((KE, 16), jnp.float32),
            pltpu.VMEM((KE, 16), jnp.float32),
            pltpu.VMEM_SHARED((NP, 16), jnp.float32),
        ],
    )
    return f(xpadb, row_p, col_p, z16)


def _heads_body(h_ref, b0_ref, b1_ref, wvx_ref, bvx_ref, wa1_ref, ba1_ref,
                wa2_ref, ba2_ref, ws1_ref, bs1_ref, ws2_ref, bs2_ref,
                wcp_ref, bcp_ref, wbl_ref, bbl_ref, vx_o, va_o, vsa_o):
    h = h_ref[...]
    vxr = jnp.dot(h, wvx_ref[...],
                  preferred_element_type=jnp.float32) + bvx_ref[...][0]
    cf = jnp.dot(h, wcp_ref[...],
                 preferred_element_type=jnp.float32) + bcp_ref[...][0]
    bwl = jnp.dot(cf, wbl_ref[...],
                  preferred_element_type=jnp.float32) + bbl_ref[...][0]
    bw = jax.nn.sigmoid(bwl[:, 0:1])
    corr = b0_ref[0] + b1_ref[0]
    vx_o[...] = vxr - 0.1 * bw * corr
    ha = jnp.maximum(jnp.dot(h, wa1_ref[...],
                             preferred_element_type=jnp.float32)
                     + ba1_ref[...][0], 0.0)
    va_o[...] = jnp.dot(ha, wa2_ref[...],
                        preferred_element_type=jnp.float32) + ba2_ref[...][0]
    hs = jnp.maximum(jnp.dot(h, ws1_ref[...],
                             preferred_element_type=jnp.float32)
                     + bs1_ref[...][0], 0.0)
    vs = jnp.dot(hs, ws2_ref[...],
                 preferred_element_type=jnp.float32) + bs2_ref[...][0]
    lane = lax.broadcasted_iota(jnp.int32, vs.shape, 1)
    vsa_o[...] = vs + (lane == NBT).astype(jnp.float32)


def _heads(h, bacc, wvx16, bvx16, wa1, ba1, wa2_32, ba2_32, ws1, bs1,
           ws2_64, bs2_64, wcp, bcp, wbl16, bbl16):
    bn = 512
    return pl.pallas_call(
        _heads_body,
        grid=(NP // bn,),
        in_specs=[
            pl.BlockSpec((bn, HID), lambda i: (i, 0)),
            pl.BlockSpec((1, bn, 16), lambda i: (0, i, 0)),
            pl.BlockSpec((1, bn, 16), lambda i: (1, i, 0)),
            pl.BlockSpec((HID, 16), lambda i: (0, 0)),
            pl.BlockSpec((1, 16), lambda i: (0, 0)),
            pl.BlockSpec((HID, HID), lambda i: (0, 0)),
            pl.BlockSpec((1, HID), lambda i: (0, 0)),
            pl.BlockSpec((HID, 32), lambda i: (0, 0)),
            pl.BlockSpec((1, 32), lambda i: (0, 0)),
            pl.BlockSpec((HID, HID), lambda i: (0, 0)),
            pl.BlockSpec((1, HID), lambda i: (0, 0)),
            pl.BlockSpec((HID, 64), lambda i: (0, 0)),
            pl.BlockSpec((1, 64), lambda i: (0, 0)),
            pl.BlockSpec((HID, HID // 2), lambda i: (0, 0)),
            pl.BlockSpec((1, HID // 2), lambda i: (0, 0)),
            pl.BlockSpec((HID // 2, 16), lambda i: (0, 0)),
            pl.BlockSpec((1, 16), lambda i: (0, 0)),
        ],
        out_specs=[
            pl.BlockSpec((bn, 16), lambda i: (i, 0)),
            pl.BlockSpec((bn, 32), lambda i: (i, 0)),
            pl.BlockSpec((bn, 64), lambda i: (i, 0)),
        ],
        out_shape=[
            jax.ShapeDtypeStruct((NP, 16), jnp.float32),
            jax.ShapeDtypeStruct((NP, 32), jnp.float32),
            jax.ShapeDtypeStruct((NP, 64), jnp.float32),
        ],
    )(h, bacc, bacc, wvx16, bvx16, wa1, ba1, wa2_32, ba2_32, ws1, bs1,
      ws2_64, bs2_64, wcp, bcp, wbl16, bbl16)


def _vs_body(vsa_hbm, bid_hbm, z64_hbm, acc_hbm, idxb, buf, accsh):
    cid = lax.axis_index("c")
    sid = lax.axis_index("s")
    per_s = (NP // NC) // NS
    nchunk = per_s // 64
    rows_per_s = NBP // NS

    pltpu.sync_copy(z64_hbm, accsh.at[pl.ds(sid * rows_per_s, rows_per_s)])
    plsc.subcore_barrier()

    @pl.loop(0, nchunk)
    def _chunk(k):
        base = cid * (NP // NC) + sid * per_s + k * 64
        pltpu.sync_copy(bid_hbm.at[pl.ds(base, 64)], idxb)
        pltpu.sync_copy(vsa_hbm.at[pl.ds(base, 64)], buf)
        pltpu.sync_copy(buf, accsh.at[idxb], add=True)

    plsc.subcore_barrier()
    pltpu.sync_copy(accsh.at[pl.ds(sid * rows_per_s, rows_per_s)],
                    acc_hbm.at[cid, pl.ds(sid * rows_per_s, rows_per_s)])


def _vs_scatter(vsa, bid_p, z64):
    f = pl.kernel(
        _vs_body,
        out_type=jax.ShapeDtypeStruct((NC, NBP, 64), jnp.float32),
        mesh=_mesh,
        compiler_params=_SC_PARAMS,
        scratch_types=[
            pltpu.VMEM((64,), jnp.int32),
            pltpu.VMEM((64, 64), jnp.float32),
            pltpu.VMEM_SHARED((NBP, 64), jnp.float32),
        ],
    )
    return f(vsa, bid_p, z64)


def _vsf_body(a0_ref, a1_ref, o_ref):
    s = a0_ref[0] + a1_ref[0]
    cnt = jnp.maximum(s[:, NBT:NBT + 1], 1.0)
    o_ref[...] = s / cnt


def _vs_final(acc):
    return pl.pallas_call(
        _vsf_body,
        grid=(1,),
        in_specs=[
            pl.BlockSpec((1, NBP, 64), lambda i: (0, 0, 0)),
            pl.BlockSpec((1, NBP, 64), lambda i: (1, 0, 0)),
        ],
        out_specs=pl.BlockSpec((NBP, 64), lambda i: (0, 0)),
        out_shape=jax.ShapeDtypeStruct((NBP, 64), jnp.float32),
    )(acc, acc)


def kernel(x_t, t, atom_embed, block_embed, edge_embed, W_e2h, b_e2h, W_tp,
           b_tp, Wms1, Wmd1, Wme1, bm1, Wua1, Wub1, bu1, Wc1, Wms2, Wmd2,
           Wme2, bm2, Wua2, Wub2, bu2, Wc2, W_vx, b_vx, Wa1, ba1, Wa2, ba2,
           Ws1, bs1, Ws2, bs2, W_cp, b_cp, W_bl, b_bl, a_t, s_t, block_ids,
           batch_ids, edges, edge_type, generate_mask):
    i32 = jnp.int32
    at_p = jnp.pad(a_t.astype(i32), (0, NP - N))
    bid_p = jnp.pad(block_ids.astype(i32), (0, NP - N),
                    constant_values=NBP - 1)
    s_t2 = jnp.pad(s_t.astype(i32), (0, NBP - NB))[:, None]
    batch2 = jnp.pad(batch_ids.astype(i32), (0, NBP - NB))[:, None]
    t2 = jnp.pad(t, (0, 16 - B))[:, None]
    row_p = jnp.pad(edges[0].astype(i32), (0, EP - E), constant_values=NP - 1)
    col_p = jnp.pad(edges[1].astype(i32), (0, EP - E), constant_values=NP - 1)
    ety_p = jnp.pad(edge_type.astype(i32), (0, EP - E))
    idx3 = jnp.stack([row_p, col_p, ety_p])
    xpad0 = jnp.pad(x_t, ((0, NP - N), (0, 13)))
    gmf = generate_mask.astype(jnp.float32)[:, None]
    gmtab = jnp.pad(gmf, ((0, NBP - NB), (3, 12)))
    z128 = jnp.zeros((NP // NS, HALF), jnp.float32)
    z16 = jnp.zeros((NP // NS, 16), jnp.float32)
    z64 = jnp.zeros((NBP // NS, 64), jnp.float32)
    wvx16 = jnp.pad(W_vx, ((0, 0), (0, 13)))
    bvx16 = jnp.pad(b_vx, (0, 13))[None]
    wa2_32 = jnp.pad(Wa2, ((0, 0), (0, 2)))
    ba2_32 = jnp.pad(ba2, (0, 2))[None]
    ws2_64 = jnp.pad(Ws2, ((0, 0), (0, 64 - NBT)))
    bs2_64 = jnp.pad(bs2, (0, 64 - NBT))[None]
    wbl16 = jnp.pad(W_bl, ((0, 0), (0, 15)))
    bbl16 = jnp.pad(b_bl, (0, 15))[None]
    permh = np.empty(HALF, np.int32)
    for g in range(4):
        permh[g * 32:g * 32 + 16] = g * 32 + 2 * np.arange(16)
        permh[g * 32 + 16:g * 32 + 32] = g * 32 + 2 * np.arange(16) + 1
    permg = np.concatenate([permh, HALF + permh])
    wub1p = Wub1[permg]
    wub2p = Wub2[permg]

    ae2, bet, et1, et2 = _tables(atom_embed, W_e2h, block_embed, s_t2,
                                 batch2, t2, b_e2h[None], b_tp[None], W_tp,
                                 edge_embed, Wme1, bm1[None], Wme2, bm2[None])
    et1_16 = et1.reshape(8, 2, HALF).transpose(1, 0, 2).reshape(16, HALF)
    et2_16 = et2.reshape(8, 2, HALF).transpose(1, 0, 2).reshape(16, HALF)

    h0, xpadb = _enc(ae2, bet, at_p, bid_p, xpad0, gmtab)

    bacc = _bond_stage(xpadb, row_p, col_p, z16)

    hs, hd = _premul(h0, Wms1, Wmd1, et1_16)
    agg1 = _edge_stage(hs.reshape(NC * NP, HALF),
                       hd.reshape(NC * 3 * NP, HALF), idx3, z128)
    h1 = _hupdate(h0, agg1, Wua1, wub1p, bu1[None])

    hs2, hd2 = _premul(h1, Wms2, Wmd2, et2_16)
    agg2 = _edge_stage(hs2.reshape(NC * NP, HALF),
                       hd2.reshape(NC * 3 * NP, HALF), idx3, z128)
    h2 = _hupdate(h1, agg2, Wua2, wub2p, bu2[None])

    vx16, va32, vsa = _heads(h2, bacc, wvx16, bvx16, Wa1, ba1[None], wa2_32,
                             ba2_32, Ws1, bs1[None], ws2_64, bs2_64,
                             W_cp, b_cp[None], wbl16, bbl16)
    vs_acc = _vs_scatter(vsa, bid_p, z64)
    vs64 = _vs_final(vs_acc)

    v_x = vx16[:N, :3]
    v_a = va32[:N, :NAT]
    v_s = vs64[:NB, :NBT]
    return (v_x, v_a, v_s)


# speedup vs baseline: 1.1034x; 1.1034x over previous
import numpy as np

import jax
import jax.numpy as jnp
from jax import lax
from jax.experimental import pallas as pl
from jax.experimental.pallas import tpu as pltpu
from jax.experimental.pallas import tpu_sc as plsc

N = 10000
E = 160000
NB = 2000
B = 4
HID = 256
EMB = 128
EDG = 64
NAT = 30
NBT = 50

NP = 10240
EP = 163840
NBP = 2048
NC = 2
NS = 16
KE = 80
HALF = 128

_mesh = plsc.VectorSubcoreMesh(core_axis_name="c", subcore_axis_name="s")
_SC_PARAMS = pltpu.CompilerParams(use_tc_tiling_on_sc=False,
                                  needs_layout_passes=False)


def _tables_body(atom_embed, W_e2h, block_embed, s_t2, batch2, t2, b_e2h,
                 b_tp, W_tp, edge_embed, Wme1, bm1, Wme2, bm2,
                 ae2_o, bet_o, et1_o, et2_o):
    ae2_o[...] = jnp.dot(atom_embed[...], W_e2h[...],
                         preferred_element_type=jnp.float32)
    t = t2[...][:, :1]
    half = HID // 2
    ii = lax.broadcasted_iota(jnp.int32, (1, half), 1).astype(jnp.float32)
    freqs = jnp.exp(-np.log(10000.0) * ii / (half - 1))
    ang = t * freqs
    te = jnp.concatenate([jnp.sin(ang), jnp.cos(ang)], axis=-1)
    te4 = jnp.dot(te, W_tp[...], preferred_element_type=jnp.float32)
    sids = s_t2[...]
    oh_s = (sids == lax.broadcasted_iota(jnp.int32, (NBP, NBT), 1)
            ).astype(jnp.float32)
    be = jnp.dot(oh_s, block_embed[...], preferred_element_type=jnp.float32)
    be2 = jnp.dot(be, W_e2h[...], preferred_element_type=jnp.float32)
    bids = batch2[...]
    oh_b = (bids == lax.broadcasted_iota(jnp.int32, (NBP, 16), 1)
            ).astype(jnp.float32)
    tpart = jnp.dot(oh_b, te4, preferred_element_type=jnp.float32)
    bet_o[...] = be2 + tpart + b_e2h[...][0] + b_tp[...][0]
    e1 = jnp.dot(edge_embed[...], Wme1[...],
                 preferred_element_type=jnp.float32) + bm1[...][0]
    e2 = jnp.dot(edge_embed[...], Wme2[...],
                 preferred_element_type=jnp.float32) + bm2[...][0]
    zpad = jnp.zeros((5, HID), jnp.float32)
    et1_o[...] = jnp.concatenate([e1, zpad], axis=0)
    et2_o[...] = jnp.concatenate([e2, zpad], axis=0)


def _tables(*args):
    return pl.pallas_call(
        _tables_body,
        out_shape=[
            jax.ShapeDtypeStruct((NAT, HID), jnp.float32),
            jax.ShapeDtypeStruct((NBP, HID), jnp.float32),
            jax.ShapeDtypeStruct((8, HID), jnp.float32),
            jax.ShapeDtypeStruct((8, HID), jnp.float32),
        ],
    )(*args)


def _enc_body(ae2_hbm, bet_hbm, at_hbm, bid_hbm, x0_hbm, gm_hbm,
              h0_hbm, xb_hbm, idxa, idxb, bufa, bufb, bufx, bufg):
    cid = lax.axis_index("c")
    sid = lax.axis_index("s")
    wid = sid * NC + cid
    per_w = NP // (NC * NS)
    nchunk = per_w // 64

    @pl.loop(0, nchunk)
    def _chunk(k):
        base = wid * per_w + k * 64
        pltpu.sync_copy(at_hbm.at[pl.ds(base, 64)], idxa)
        pltpu.sync_copy(bid_hbm.at[pl.ds(base, 64)], idxb)
        pltpu.sync_copy(ae2_hbm.at[idxa], bufa)
        pltpu.sync_copy(bet_hbm.at[idxb], bufb)
        pltpu.sync_copy(x0_hbm.at[pl.ds(base, 64)], bufx)
        pltpu.sync_copy(gm_hbm.at[idxb], bufg)

        @pl.loop(0, 64)
        def _row(r):
            for c in range(HID // 16):
                sl = pl.ds(c * 16, 16)
                bufa[r, sl] = bufa[r, sl] + bufb[r, sl]
            bufx[r, :] = bufx[r, :] + bufg[r, :]

        pltpu.sync_copy(bufa, h0_hbm.at[pl.ds(base, 64)])
        pltpu.sync_copy(bufx, xb_hbm.at[pl.ds(base, 64)])


def _enc(ae2, bet, at_p, bid_p, xpad0, gmtab):
    f = pl.kernel(
        _enc_body,
        out_type=[
            jax.ShapeDtypeStruct((NP, HID), jnp.float32),
            jax.ShapeDtypeStruct((NP, 16), jnp.float32),
        ],
        mesh=_mesh,
        compiler_params=_SC_PARAMS,
        scratch_types=[
            pltpu.VMEM((64,), jnp.int32),
            pltpu.VMEM((64,), jnp.int32),
            pltpu.VMEM((64, HID), jnp.float32),
            pltpu.VMEM((64, HID), jnp.float32),
            pltpu.VMEM((64, 16), jnp.float32),
            pltpu.VMEM((64, 16), jnp.float32),
        ],
    )
    return f(ae2, bet, at_p, bid_p, xpad0, gmtab)


def _premul_body(h_ref, wms_ref, wmd_ref, et_ref, hs_o, hd_o):
    hcol = pl.program_id(0)
    hblk = h_ref[...]
    ws = wms_ref[:, pl.ds(hcol * HALF, HALF)]
    wd = wmd_ref[:, pl.ds(hcol * HALF, HALF)]
    hs_o[...] = jnp.dot(hblk, ws, preferred_element_type=jnp.float32)[None]
    hd = jnp.dot(hblk, wd, preferred_element_type=jnp.float32)
    ets = et_ref[pl.ds(hcol * 8, 8)]
    for tt in range(3):
        hd_o[0, tt] = hd + ets[tt]


def _premul(h, wms, wmd, et16):
    bn = 512
    return pl.pallas_call(
        _premul_body,
        grid=(NC, NP // bn),
        in_specs=[
            pl.BlockSpec((bn, HID), lambda hh, i: (i, 0)),
            pl.BlockSpec((HID, HID), lambda hh, i: (0, 0)),
            pl.BlockSpec((HID, HID), lambda hh, i: (0, 0)),
            pl.BlockSpec((16, HALF), lambda hh, i: (0, 0)),
        ],
        out_specs=[
            pl.BlockSpec((1, bn, HALF), lambda hh, i: (hh, i, 0)),
            pl.BlockSpec((1, 3, bn, HALF), lambda hh, i: (hh, 0, i, 0)),
        ],
        out_shape=[
            jax.ShapeDtypeStruct((NC, NP, HALF), jnp.float32),
            jax.ShapeDtypeStruct((NC, 3, NP, HALF), jnp.float32),
        ],
    )(h, wms, wmd, et16)


def _edge_body(hsf_hbm, hdf_hbm, idx3_hbm, z128_hbm, agg_hbm,
               idxr2_0, idxc2_0, bufs_0, bufd_0, idxs_0,
               idxr2_1, idxc2_1, bufs_1, bufd_1, idxs_1,
               ib_0, ib_1,
               semg0, semg1, sems0, sems1, semi0, semi1, aggsh):
    cid = lax.axis_index("c")
    sid = lax.axis_index("s")
    coff = cid * NP
    coffd = cid * 3 * NP
    per_s = EP // NS
    nchunk = per_s // KE
    npair = nchunk // 2
    rows_per_s = NP // NS

    pltpu.sync_copy(z128_hbm, aggsh.at[pl.ds(sid * rows_per_s, rows_per_s)])
    plsc.subcore_barrier()

    sets = ((idxr2_0, idxc2_0, bufs_0, bufd_0, idxs_0, semg0, sems0),
            (idxr2_1, idxc2_1, bufs_1, bufd_1, idxs_1, semg1, sems1))
    ibufs = ((ib_0, semi0), (ib_1, semi1))

    def idx_copy(pair, pb):
        ib, semi = ibufs[pb]
        base = sid * per_s + pair * 2 * KE
        return pltpu.make_async_copy(
            idx3_hbm.at[:, pl.ds(base, 2 * KE)], ib, semi)

    def scatter_wait(b):
        _, _, bufs, _, idxs, _, sems = sets[b]
        pltpu.make_async_copy(bufs, aggsh.at[idxs.at[0]], sems).wait()

    def issue_gathers(b, pb):
        idxr2, idxc2, bufs, bufd, _, semg, _ = sets[b]
        ib, _ = ibufs[pb]
        off = b * KE
        for j in range(KE // 16):
            sl = pl.ds(j * 16, 16)
            si = pl.ds(off + j * 16, 16)
            idxr2[sl] = ib[0, si] + coff
            idxc2[sl] = ib[1, si] + ib[2, si] * NP + coffd
        pltpu.async_copy(hsf_hbm.at[idxr2], bufs, semg)
        pltpu.async_copy(hdf_hbm.at[idxc2], bufd, semg)

    def wait_gather(b):
        idxr2, idxc2, bufs, bufd, _, semg, _ = sets[b]
        pltpu.make_async_copy(hsf_hbm.at[idxr2], bufs, semg).wait()
        pltpu.make_async_copy(hdf_hbm.at[idxc2], bufd, semg).wait()

    idx_copy(0, 0).start()
    idx_copy(0, 0).wait()
    issue_gathers(0, 0)
    issue_gathers(1, 0)
    idx_copy(1, 1).start()

    @pl.loop(0, npair, step=2)
    def _quad(q):
        for pp in range(2):
            pair = q + pp
            ib, _ = ibufs[pp]
            for b in range(2):
                _, _, bufs, bufd, idxs, _, sems = sets[b]
                wait_gather(b)

                @pl.loop(0, KE)
                def _edge(e):
                    for c in range(8):
                        sl = pl.ds(c * 16, 16)
                        bufs[e, sl] = jnp.maximum(
                            bufs[e, sl] + bufd[e, sl], 0.0)

                for j in range(KE // 16):
                    sl = pl.ds(j * 16, 16)
                    idxs[0, sl] = ib[0, pl.ds(b * KE + j * 16, 16)]
                pltpu.async_copy(bufs, aggsh.at[idxs.at[0]], sems, add=True)

                if b == 0:
                    @pl.when(pair + 1 < npair)
                    def _():
                        idx_copy(pair + 1, pp ^ 1).wait()

                @pl.when(pair + 1 < npair)
                def _():
                    scatter_wait(b)
                    issue_gathers(b, pp ^ 1)

                if b == 1:
                    @pl.when(pair + 2 < npair)
                    def _():
                        idx_copy(pair + 2, pp).start()

    scatter_wait(0)
    scatter_wait(1)
    plsc.subcore_barrier()
    pltpu.sync_copy(aggsh.at[pl.ds(sid * rows_per_s, rows_per_s)],
                    agg_hbm.at[cid, pl.ds(sid * rows_per_s, rows_per_s)])


def _edge_stage(hsf, hdf, idx3, z128):
    bufset = [
        pltpu.VMEM((KE,), jnp.int32),
        pltpu.VMEM((KE,), jnp.int32),
        pltpu.VMEM((KE, HALF), jnp.float32),
        pltpu.VMEM((KE, HALF), jnp.float32),
        pltpu.VMEM((1, KE), jnp.int32),
    ]
    f = pl.kernel(
        _edge_body,
        out_type=jax.ShapeDtypeStruct((NC, NP, HALF), jnp.float32),
        mesh=_mesh,
        compiler_params=_SC_PARAMS,
        scratch_types=bufset + bufset + [
            pltpu.VMEM((3, 2 * KE), jnp.int32),
            pltpu.VMEM((3, 2 * KE), jnp.int32),
            pltpu.SemaphoreType.DMA,
            pltpu.SemaphoreType.DMA,
            pltpu.SemaphoreType.DMA,
            pltpu.SemaphoreType.DMA,
            pltpu.SemaphoreType.DMA,
            pltpu.SemaphoreType.DMA,
        ] + [
            pltpu.VMEM_SHARED((NP, HALF), jnp.float32),
        ],
    )
    return f(hsf, hdf, idx3, z128)


def _hupd_body(h_ref, a0_ref, a1_ref, wua_ref, wub_ref, bu_ref, o_ref):
    h = h_ref[...]
    z = jnp.dot(h, wua_ref[...], preferred_element_type=jnp.float32)
    z += jnp.dot(a0_ref[0].astype(jnp.float32), wub_ref[...][:HALF, :],
                 preferred_element_type=jnp.float32)
    z += jnp.dot(a1_ref[0].astype(jnp.float32), wub_ref[...][HALF:, :],
                 preferred_element_type=jnp.float32)
    o_ref[...] = h + jnp.maximum(z + bu_ref[...][0], 0.0)


def _hupdate(h, agg, wua, wub, bu):
    bn = 512
    return pl.pallas_call(
        _hupd_body,
        grid=(NP // bn,),
        in_specs=[
            pl.BlockSpec((bn, HID), lambda i: (i, 0)),
            pl.BlockSpec((1, bn, HALF), lambda i: (0, i, 0)),
            pl.BlockSpec((1, bn, HALF), lambda i: (1, i, 0)),
            pl.BlockSpec((HID, HID), lambda i: (0, 0)),
            pl.BlockSpec((HID, HID), lambda i: (0, 0)),
            pl.BlockSpec((1, HID), lambda i: (0, 0)),
        ],
        out_specs=pl.BlockSpec((bn, HID), lambda i: (i, 0)),
        out_shape=jax.ShapeDtypeStruct((NP, HID), jnp.float32),
    )(h, agg, agg, wua, wub, bu)


def _bond_body(xb_hbm, row_hbm, col_hbm, z16_hbm, bacc_hbm,
               idxr, idxc, xr, xc, outr, outc, accsh):
    cid = lax.axis_index("c")
    sid = lax.axis_index("s")
    per_s = (EP // NC) // NS
    nchunk = per_s // KE
    rows_per_s = NP // NS

    pltpu.sync_copy(z16_hbm, accsh.at[pl.ds(sid * rows_per_s, rows_per_s)])
    plsc.subcore_barrier()

    lane = lax.iota(jnp.int32, 16)
    zeros16 = jnp.zeros((16,), jnp.float32)

    @pl.loop(0, nchunk)
    def _chunk(k):
        base = cid * (EP // NC) + sid * per_s + k * KE
        pltpu.sync_copy(row_hbm.at[pl.ds(base, KE)], idxr)
        pltpu.sync_copy(col_hbm.at[pl.ds(base, KE)], idxc)
        pltpu.sync_copy(xb_hbm.at[idxr], xr)
        pltpu.sync_copy(xb_hbm.at[idxc], xc)

        @pl.loop(0, KE)
        def _edge(e):
            a = xr[e, :]
            b = xc[e, :]
            d = jnp.where(lane < 3, a - b, zeros16)
            s = jnp.maximum(jnp.sum(d * d), 1e-30)
            act = jnp.sum(jnp.where(lane == 3, jnp.maximum(a, b), zeros16))
            sv = jnp.full((16,), s, jnp.float32)
            ri = jnp.int32(0x5F3759DF) - (plsc.bitcast(sv, jnp.int32) >> 1)
            r = plsc.bitcast(ri, jnp.float32)
            r = r * (1.5 - 0.5 * sv * r * r)
            r = r * (1.5 - 0.5 * sv * r * r)
            r = r * (1.5 - 0.5 * sv * r * r)
            dist = sv * r + 1e-8
            inv = 1.0 / dist
            contrib = (act * d) * ((dist - 1.5) * inv * inv)
            outr[e, :] = contrib
            outc[e, :] = -contrib

        pltpu.sync_copy(outr, accsh.at[idxr], add=True)
        pltpu.sync_copy(outc, accsh.at[idxc], add=True)

    plsc.subcore_barrier()
    pltpu.sync_copy(accsh.at[pl.ds(sid * rows_per_s, rows_per_s)],
                    bacc_hbm.at[cid, pl.ds(sid * rows_per_s, rows_per_s)])


def _bond_stage(xpadb, row_p, col_p, z16):
    f = pl.kernel(
        _bond_body,
        out_type=jax.ShapeDtypeStruct((NC, NP, 16), jnp.float32),
        mesh=_mesh,
        compiler_params=_SC_PARAMS,
        scratch_types=[
            pltpu.VMEM((KE,), jnp.int32),
            pltpu.VMEM((KE,), jnp.int32),
            pltpu.VMEM((KE, 16), jnp.float32),
            pltpu.VMEM((KE, 16), jnp.float32),
            pltpu.VMEM((KE, 16), jnp.float32),
            pltpu.VMEM((KE, 16), jnp.float32),
            pltpu.VMEM_SHARED((NP, 16), jnp.float32),
        ],
    )
    return f(xpadb, row_p, col_p, z16)


def _heads_body(h_ref, b0_ref, b1_ref, wvx_ref, bvx_ref, wa1_ref, ba1_ref,
                wa2_ref, ba2_ref, ws1_ref, bs1_ref, ws2_ref, bs2_ref,
                wcp_ref, bcp_ref, wbl_ref, bbl_ref, vx_o, va_o, vsa_o):
    h = h_ref[...]
    vxr = jnp.dot(h, wvx_ref[...],
                  preferred_element_type=jnp.float32) + bvx_ref[...][0]
    cf = jnp.dot(h, wcp_ref[...],
                 preferred_element_type=jnp.float32) + bcp_ref[...][0]
    bwl = jnp.dot(cf, wbl_ref[...],
                  preferred_element_type=jnp.float32) + bbl_ref[...][0]
    bw = jax.nn.sigmoid(bwl[:, 0:1])
    corr = b0_ref[0] + b1_ref[0]
    vx_o[...] = vxr - 0.1 * bw * corr
    ha = jnp.maximum(jnp.dot(h, wa1_ref[...],
                             preferred_element_type=jnp.float32)
                     + ba1_ref[...][0], 0.0)
    va_o[...] = jnp.dot(ha, wa2_ref[...],
                        preferred_element_type=jnp.float32) + ba2_ref[...][0]
    hs = jnp.maximum(jnp.dot(h, ws1_ref[...],
                             preferred_element_type=jnp.float32)
                     + bs1_ref[...][0], 0.0)
    vs = jnp.dot(hs, ws2_ref[...],
                 preferred_element_type=jnp.float32) + bs2_ref[...][0]
    lane = lax.broadcasted_iota(jnp.int32, vs.shape, 1)
    vsa_o[...] = vs + (lane == NBT).astype(jnp.float32)


def _heads(h, bacc, wvx16, bvx16, wa1, ba1, wa2_32, ba2_32, ws1, bs1,
           ws2_64, bs2_64, wcp, bcp, wbl16, bbl16):
    bn = 512
    return pl.pallas_call(
        _heads_body,
        grid=(NP // bn,),
        in_specs=[
            pl.BlockSpec((bn, HID), lambda i: (i, 0)),
            pl.BlockSpec((1, bn, 16), lambda i: (0, i, 0)),
            pl.BlockSpec((1, bn, 16), lambda i: (1, i, 0)),
            pl.BlockSpec((HID, 16), lambda i: (0, 0)),
            pl.BlockSpec((1, 16), lambda i: (0, 0)),
            pl.BlockSpec((HID, HID), lambda i: (0, 0)),
            pl.BlockSpec((1, HID), lambda i: (0, 0)),
            pl.BlockSpec((HID, 32), lambda i: (0, 0)),
            pl.BlockSpec((1, 32), lambda i: (0, 0)),
            pl.BlockSpec((HID, HID), lambda i: (0, 0)),
            pl.BlockSpec((1, HID), lambda i: (0, 0)),
            pl.BlockSpec((HID, 64), lambda i: (0, 0)),
            pl.BlockSpec((1, 64), lambda i: (0, 0)),
            pl.BlockSpec((HID, HID // 2), lambda i: (0, 0)),
            pl.BlockSpec((1, HID // 2), lambda i: (0, 0)),
            pl.BlockSpec((HID // 2, 16), lambda i: (0, 0)),
            pl.BlockSpec((1, 16), lambda i: (0, 0)),
        ],
        out_specs=[
            pl.BlockSpec((bn, 16), lambda i: (i, 0)),
            pl.BlockSpec((bn, 32), lambda i: (i, 0)),
            pl.BlockSpec((bn, 64), lambda i: (i, 0)),
        ],
        out_shape=[
            jax.ShapeDtypeStruct((NP, 16), jnp.float32),
            jax.ShapeDtypeStruct((NP, 32), jnp.float32),
            jax.ShapeDtypeStruct((NP, 64), jnp.float32),
        ],
    )(h, bacc, bacc, wvx16, bvx16, wa1, ba1, wa2_32, ba2_32, ws1, bs1,
      ws2_64, bs2_64, wcp, bcp, wbl16, bbl16)


def _vs_body(vsa_hbm, bid_hbm, z64_hbm, acc_hbm, idxb, buf, accsh):
    cid = lax.axis_index("c")
    sid = lax.axis_index("s")
    per_s = (NP // NC) // NS
    nchunk = per_s // 64
    rows_per_s = NBP // NS

    pltpu.sync_copy(z64_hbm, accsh.at[pl.ds(sid * rows_per_s, rows_per_s)])
    plsc.subcore_barrier()

    @pl.loop(0, nchunk)
    def _chunk(k):
        base = cid * (NP // NC) + sid * per_s + k * 64
        pltpu.sync_copy(bid_hbm.at[pl.ds(base, 64)], idxb)
        pltpu.sync_copy(vsa_hbm.at[pl.ds(base, 64)], buf)
        pltpu.sync_copy(buf, accsh.at[idxb], add=True)

    plsc.subcore_barrier()
    pltpu.sync_copy(accsh.at[pl.ds(sid * rows_per_s, rows_per_s)],
                    acc_hbm.at[cid, pl.ds(sid * rows_per_s, rows_per_s)])


def _vs_scatter(vsa, bid_p, z64):
    f = pl.kernel(
        _vs_body,
        out_type=jax.ShapeDtypeStruct((NC, NBP, 64), jnp.float32),
        mesh=_mesh,
        compiler_params=_SC_PARAMS,
        scratch_types=[
            pltpu.VMEM((64,), jnp.int32),
            pltpu.VMEM((64, 64), jnp.float32),
            pltpu.VMEM_SHARED((NBP, 64), jnp.float32),
        ],
    )
    return f(vsa, bid_p, z64)


def _vsf_body(a0_ref, a1_ref, o_ref):
    s = a0_ref[0] + a1_ref[0]
    cnt = jnp.maximum(s[:, NBT:NBT + 1], 1.0)
    o_ref[...] = s / cnt


def _vs_final(acc):
    return pl.pallas_call(
        _vsf_body,
        grid=(1,),
        in_specs=[
            pl.BlockSpec((1, NBP, 64), lambda i: (0, 0, 0)),
            pl.BlockSpec((1, NBP, 64), lambda i: (1, 0, 0)),
        ],
        out_specs=pl.BlockSpec((NBP, 64), lambda i: (0, 0)),
        out_shape=jax.ShapeDtypeStruct((NBP, 64), jnp.float32),
    )(acc, acc)


def kernel(x_t, t, atom_embed, block_embed, edge_embed, W_e2h, b_e2h, W_tp,
           b_tp, Wms1, Wmd1, Wme1, bm1, Wua1, Wub1, bu1, Wc1, Wms2, Wmd2,
           Wme2, bm2, Wua2, Wub2, bu2, Wc2, W_vx, b_vx, Wa1, ba1, Wa2, ba2,
           Ws1, bs1, Ws2, bs2, W_cp, b_cp, W_bl, b_bl, a_t, s_t, block_ids,
           batch_ids, edges, edge_type, generate_mask):
    i32 = jnp.int32
    at_p = jnp.pad(a_t.astype(i32), (0, NP - N))
    bid_p = jnp.pad(block_ids.astype(i32), (0, NP - N),
                    constant_values=NBP - 1)
    s_t2 = jnp.pad(s_t.astype(i32), (0, NBP - NB))[:, None]
    batch2 = jnp.pad(batch_ids.astype(i32), (0, NBP - NB))[:, None]
    t2 = jnp.pad(t, (0, 16 - B))[:, None]
    row_p = jnp.pad(edges[0].astype(i32), (0, EP - E), constant_values=NP - 1)
    col_p = jnp.pad(edges[1].astype(i32), (0, EP - E), constant_values=NP - 1)
    ety_p = jnp.pad(edge_type.astype(i32), (0, EP - E))
    idx3 = jnp.stack([row_p, col_p, ety_p])
    xpad0 = jnp.pad(x_t, ((0, NP - N), (0, 13)))
    gmf = generate_mask.astype(jnp.float32)[:, None]
    gmtab = jnp.pad(gmf, ((0, NBP - NB), (3, 12)))
    z128 = jnp.zeros((NP // NS, HALF), jnp.float32)
    z16 = jnp.zeros((NP // NS, 16), jnp.float32)
    z64 = jnp.zeros((NBP // NS, 64), jnp.float32)
    wvx16 = jnp.pad(W_vx, ((0, 0), (0, 13)))
    bvx16 = jnp.pad(b_vx, (0, 13))[None]
    wa2_32 = jnp.pad(Wa2, ((0, 0), (0, 2)))
    ba2_32 = jnp.pad(ba2, (0, 2))[None]
    ws2_64 = jnp.pad(Ws2, ((0, 0), (0, 64 - NBT)))
    bs2_64 = jnp.pad(bs2, (0, 64 - NBT))[None]
    wbl16 = jnp.pad(W_bl, ((0, 0), (0, 15)))
    bbl16 = jnp.pad(b_bl, (0, 15))[None]
    wub1p = Wub1
    wub2p = Wub2

    ae2, bet, et1, et2 = _tables(atom_embed, W_e2h, block_embed, s_t2,
                                 batch2, t2, b_e2h[None], b_tp[None], W_tp,
                                 edge_embed, Wme1, bm1[None], Wme2, bm2[None])
    et1_16 = et1.reshape(8, 2, HALF).transpose(1, 0, 2).reshape(16, HALF)
    et2_16 = et2.reshape(8, 2, HALF).transpose(1, 0, 2).reshape(16, HALF)

    h0, xpadb = _enc(ae2, bet, at_p, bid_p, xpad0, gmtab)

    bacc = _bond_stage(xpadb, row_p, col_p, z16)

    hs, hd = _premul(h0, Wms1, Wmd1, et1_16)
    agg1 = _edge_stage(hs.reshape(NC * NP, HALF),
                       hd.reshape(NC * 3 * NP, HALF), idx3, z128)
    h1 = _hupdate(h0, agg1, Wua1, wub1p, bu1[None])

    hs2, hd2 = _premul(h1, Wms2, Wmd2, et2_16)
    agg2 = _edge_stage(hs2.reshape(NC * NP, HALF),
                       hd2.reshape(NC * 3 * NP, HALF), idx3, z128)
    h2 = _hupdate(h1, agg2, Wua2, wub2p, bu2[None])

    vx16, va32, vsa = _heads(h2, bacc, wvx16, bvx16, Wa1, ba1[None], wa2_32,
                             ba2_32, Ws1, bs1[None], ws2_64, bs2_64,
                             W_cp, b_cp[None], wbl16, bbl16)
    vs_acc = _vs_scatter(vsa, bid_p, z64)
    vs64 = _vs_final(vs_acc)

    v_x = vx16[:N, :3]
    v_a = va32[:N, :NAT]
    v_s = vs64[:NB, :NBT]
    return (v_x, v_a, v_s)


# speedup vs baseline: 1.1165x; 1.0119x over previous
import numpy as np

import jax
import jax.numpy as jnp
from jax import lax
from jax.experimental import pallas as pl
from jax.experimental.pallas import tpu as pltpu
from jax.experimental.pallas import tpu_sc as plsc

N = 10000
E = 160000
NB = 2000
B = 4
HID = 256
EMB = 128
EDG = 64
NAT = 30
NBT = 50

NP = 10240
EP = 163840
NBP = 2048
NC = 2
NS = 16
KE = 80
HALF = 128

_mesh = plsc.VectorSubcoreMesh(core_axis_name="c", subcore_axis_name="s")
_SC_PARAMS = pltpu.CompilerParams(use_tc_tiling_on_sc=False,
                                  needs_layout_passes=False)


def _tables_body(atom_embed, W_e2h, block_embed, s_t2, batch2, t2, b_e2h,
                 b_tp, W_tp, edge_embed, Wme1, bm1, Wme2, bm2,
                 ae2_o, bet_o, et1_o, et2_o):
    ae2_o[...] = jnp.dot(atom_embed[...], W_e2h[...],
                         preferred_element_type=jnp.float32)
    t = t2[...][:, :1]
    half = HID // 2
    ii = lax.broadcasted_iota(jnp.int32, (1, half), 1).astype(jnp.float32)
    freqs = jnp.exp(-np.log(10000.0) * ii / (half - 1))
    ang = t * freqs
    te = jnp.concatenate([jnp.sin(ang), jnp.cos(ang)], axis=-1)
    te4 = jnp.dot(te, W_tp[...], preferred_element_type=jnp.float32)
    sids = s_t2[...]
    oh_s = (sids == lax.broadcasted_iota(jnp.int32, (NBP, NBT), 1)
            ).astype(jnp.float32)
    be = jnp.dot(oh_s, block_embed[...], preferred_element_type=jnp.float32)
    be2 = jnp.dot(be, W_e2h[...], preferred_element_type=jnp.float32)
    bids = batch2[...]
    oh_b = (bids == lax.broadcasted_iota(jnp.int32, (NBP, 16), 1)
            ).astype(jnp.float32)
    tpart = jnp.dot(oh_b, te4, preferred_element_type=jnp.float32)
    bet_o[...] = be2 + tpart + b_e2h[...][0] + b_tp[...][0]
    e1 = jnp.dot(edge_embed[...], Wme1[...],
                 preferred_element_type=jnp.float32) + bm1[...][0]
    e2 = jnp.dot(edge_embed[...], Wme2[...],
                 preferred_element_type=jnp.float32) + bm2[...][0]
    zpad = jnp.zeros((5, HID), jnp.float32)
    et1_o[...] = jnp.concatenate([e1, zpad], axis=0)
    et2_o[...] = jnp.concatenate([e2, zpad], axis=0)


def _tables(*args):
    return pl.pallas_call(
        _tables_body,
        out_shape=[
            jax.ShapeDtypeStruct((NAT, HID), jnp.float32),
            jax.ShapeDtypeStruct((NBP, HID), jnp.float32),
            jax.ShapeDtypeStruct((8, HID), jnp.float32),
            jax.ShapeDtypeStruct((8, HID), jnp.float32),
        ],
    )(*args)


def _enc_body(ae2_hbm, bet_hbm, at_hbm, bid_hbm, x0_hbm, gm_hbm,
              h0_hbm, xb_hbm, idxa, idxb, bufa, bufb, bufx, bufg):
    cid = lax.axis_index("c")
    sid = lax.axis_index("s")
    wid = sid * NC + cid
    per_w = NP // (NC * NS)
    nchunk = per_w // 64

    @pl.loop(0, nchunk)
    def _chunk(k):
        base = wid * per_w + k * 64
        pltpu.sync_copy(at_hbm.at[pl.ds(base, 64)], idxa)
        pltpu.sync_copy(bid_hbm.at[pl.ds(base, 64)], idxb)
        pltpu.sync_copy(ae2_hbm.at[idxa], bufa)
        pltpu.sync_copy(bet_hbm.at[idxb], bufb)
        pltpu.sync_copy(x0_hbm.at[pl.ds(base, 64)], bufx)
        pltpu.sync_copy(gm_hbm.at[idxb], bufg)

        @plsc.parallel_loop(0, 64, unroll=4)
        def _row(r):
            for c in range(HID // 16):
                sl = pl.ds(c * 16, 16)
                bufa[r, sl] = bufa[r, sl] + bufb[r, sl]
            bufx[r, :] = bufx[r, :] + bufg[r, :]

        pltpu.sync_copy(bufa, h0_hbm.at[pl.ds(base, 64)])
        pltpu.sync_copy(bufx, xb_hbm.at[pl.ds(base, 64)])


def _enc(ae2, bet, at_p, bid_p, xpad0, gmtab):
    f = pl.kernel(
        _enc_body,
        out_type=[
            jax.ShapeDtypeStruct((NP, HID), jnp.float32),
            jax.ShapeDtypeStruct((NP, 16), jnp.float32),
        ],
        mesh=_mesh,
        compiler_params=_SC_PARAMS,
        scratch_types=[
            pltpu.VMEM((64,), jnp.int32),
            pltpu.VMEM((64,), jnp.int32),
            pltpu.VMEM((64, HID), jnp.float32),
            pltpu.VMEM((64, HID), jnp.float32),
            pltpu.VMEM((64, 16), jnp.float32),
            pltpu.VMEM((64, 16), jnp.float32),
        ],
    )
    return f(ae2, bet, at_p, bid_p, xpad0, gmtab)


def _premul_body(h_ref, wms_ref, wmd_ref, et_ref, hs_o, hd_o):
    hcol = pl.program_id(0)
    hblk = h_ref[...]
    ws = wms_ref[:, pl.ds(hcol * HALF, HALF)]
    wd = wmd_ref[:, pl.ds(hcol * HALF, HALF)]
    hs_o[...] = jnp.dot(hblk, ws, preferred_element_type=jnp.float32)[None]
    hd = jnp.dot(hblk, wd, preferred_element_type=jnp.float32)
    ets = et_ref[pl.ds(hcol * 8, 8)]
    for tt in range(3):
        hd_o[0, tt] = hd + ets[tt]


def _premul(h, wms, wmd, et16):
    bn = 512
    return pl.pallas_call(
        _premul_body,
        grid=(NC, NP // bn),
        in_specs=[
            pl.BlockSpec((bn, HID), lambda hh, i: (i, 0)),
            pl.BlockSpec((HID, HID), lambda hh, i: (0, 0)),
            pl.BlockSpec((HID, HID), lambda hh, i: (0, 0)),
            pl.BlockSpec((16, HALF), lambda hh, i: (0, 0)),
        ],
        out_specs=[
            pl.BlockSpec((1, bn, HALF), lambda hh, i: (hh, i, 0)),
            pl.BlockSpec((1, 3, bn, HALF), lambda hh, i: (hh, 0, i, 0)),
        ],
        out_shape=[
            jax.ShapeDtypeStruct((NC, NP, HALF), jnp.float32),
            jax.ShapeDtypeStruct((NC, 3, NP, HALF), jnp.float32),
        ],
    )(h, wms, wmd, et16)


def _edge_body(hsf_hbm, hdf_hbm, idx3_hbm, z128_hbm, agg_hbm,
               idxr2_0, idxc2_0, bufs_0, bufd_0, idxs_0,
               idxr2_1, idxc2_1, bufs_1, bufd_1, idxs_1,
               ib_0, ib_1,
               semg0, semg1, sems0, sems1, semi0, semi1, aggsh):
    cid = lax.axis_index("c")
    sid = lax.axis_index("s")
    coff = cid * NP
    coffd = cid * 3 * NP
    per_s = EP // NS
    nchunk = per_s // KE
    npair = nchunk // 2
    rows_per_s = NP // NS

    pltpu.sync_copy(z128_hbm, aggsh.at[pl.ds(sid * rows_per_s, rows_per_s)])
    plsc.subcore_barrier()

    sets = ((idxr2_0, idxc2_0, bufs_0, bufd_0, idxs_0, semg0, sems0),
            (idxr2_1, idxc2_1, bufs_1, bufd_1, idxs_1, semg1, sems1))
    ibufs = ((ib_0, semi0), (ib_1, semi1))

    def idx_copy(pair, pb):
        ib, semi = ibufs[pb]
        base = sid * per_s + pair * 2 * KE
        return pltpu.make_async_copy(
            idx3_hbm.at[:, pl.ds(base, 2 * KE)], ib, semi)

    def scatter_wait(b):
        _, _, bufs, _, idxs, _, sems = sets[b]
        pltpu.make_async_copy(bufs, aggsh.at[idxs.at[0]], sems).wait()

    def issue_gathers(b, pb):
        idxr2, idxc2, bufs, bufd, _, semg, _ = sets[b]
        ib, _ = ibufs[pb]
        off = b * KE
        for j in range(KE // 16):
            sl = pl.ds(j * 16, 16)
            si = pl.ds(off + j * 16, 16)
            idxr2[sl] = ib[0, si] + coff
            idxc2[sl] = ib[1, si] + ib[2, si] * NP + coffd
        pltpu.async_copy(hsf_hbm.at[idxr2], bufs, semg)
        pltpu.async_copy(hdf_hbm.at[idxc2], bufd, semg)

    def wait_gather(b):
        idxr2, idxc2, bufs, bufd, _, semg, _ = sets[b]
        pltpu.make_async_copy(hsf_hbm.at[idxr2], bufs, semg).wait()
        pltpu.make_async_copy(hdf_hbm.at[idxc2], bufd, semg).wait()

    idx_copy(0, 0).start()
    idx_copy(0, 0).wait()
    issue_gathers(0, 0)
    issue_gathers(1, 0)
    idx_copy(1, 1).start()

    @pl.loop(0, npair, step=2)
    def _quad(q):
        for pp in range(2):
            pair = q + pp
            ib, _ = ibufs[pp]
            for b in range(2):
                _, _, bufs, bufd, idxs, _, sems = sets[b]
                wait_gather(b)

                @plsc.parallel_loop(0, KE, unroll=4)
                def _edge(e):
                    for c in range(8):
                        sl = pl.ds(c * 16, 16)
                        bufs[e, sl] = jnp.maximum(
                            bufs[e, sl] + bufd[e, sl], 0.0)

                for j in range(KE // 16):
                    sl = pl.ds(j * 16, 16)
                    idxs[0, sl] = ib[0, pl.ds(b * KE + j * 16, 16)]
                pltpu.async_copy(bufs, aggsh.at[idxs.at[0]], sems, add=True)

                if b == 0:
                    @pl.when(pair + 1 < npair)
                    def _():
                        idx_copy(pair + 1, pp ^ 1).wait()

                @pl.when(pair + 1 < npair)
                def _():
                    scatter_wait(b)
                    issue_gathers(b, pp ^ 1)

                if b == 1:
                    @pl.when(pair + 2 < npair)
                    def _():
                        idx_copy(pair + 2, pp).start()

    scatter_wait(0)
    scatter_wait(1)
    plsc.subcore_barrier()
    pltpu.sync_copy(aggsh.at[pl.ds(sid * rows_per_s, rows_per_s)],
                    agg_hbm.at[cid, pl.ds(sid * rows_per_s, rows_per_s)])


def _edge_stage(hsf, hdf, idx3, z128):
    bufset = [
        pltpu.VMEM((KE,), jnp.int32),
        pltpu.VMEM((KE,), jnp.int32),
        pltpu.VMEM((KE, HALF), jnp.float32),
        pltpu.VMEM((KE, HALF), jnp.float32),
        pltpu.VMEM((1, KE), jnp.int32),
    ]
    f = pl.kernel(
        _edge_body,
        out_type=jax.ShapeDtypeStruct((NC, NP, HALF), jnp.float32),
        mesh=_mesh,
        compiler_params=_SC_PARAMS,
        scratch_types=bufset + bufset + [
            pltpu.VMEM((3, 2 * KE), jnp.int32),
            pltpu.VMEM((3, 2 * KE), jnp.int32),
            pltpu.SemaphoreType.DMA,
            pltpu.SemaphoreType.DMA,
            pltpu.SemaphoreType.DMA,
            pltpu.SemaphoreType.DMA,
            pltpu.SemaphoreType.DMA,
            pltpu.SemaphoreType.DMA,
        ] + [
            pltpu.VMEM_SHARED((NP, HALF), jnp.float32),
        ],
    )
    return f(hsf, hdf, idx3, z128)


def _hupd_body(h_ref, a0_ref, a1_ref, wua_ref, wub_ref, bu_ref, o_ref):
    h = h_ref[...]
    z = jnp.dot(h, wua_ref[...], preferred_element_type=jnp.float32)
    z += jnp.dot(a0_ref[0].astype(jnp.float32), wub_ref[...][:HALF, :],
                 preferred_element_type=jnp.float32)
    z += jnp.dot(a1_ref[0].astype(jnp.float32), wub_ref[...][HALF:, :],
                 preferred_element_type=jnp.float32)
    o_ref[...] = h + jnp.maximum(z + bu_ref[...][0], 0.0)


def _hupdate(h, agg, wua, wub, bu):
    bn = 512
    return pl.pallas_call(
        _hupd_body,
        grid=(NP // bn,),
        in_specs=[
            pl.BlockSpec((bn, HID), lambda i: (i, 0)),
            pl.BlockSpec((1, bn, HALF), lambda i: (0, i, 0)),
            pl.BlockSpec((1, bn, HALF), lambda i: (1, i, 0)),
            pl.BlockSpec((HID, HID), lambda i: (0, 0)),
            pl.BlockSpec((HID, HID), lambda i: (0, 0)),
            pl.BlockSpec((1, HID), lambda i: (0, 0)),
        ],
        out_specs=pl.BlockSpec((bn, HID), lambda i: (i, 0)),
        out_shape=jax.ShapeDtypeStruct((NP, HID), jnp.float32),
    )(h, agg, agg, wua, wub, bu)


def _bond_body(xb_hbm, row_hbm, col_hbm, z16_hbm, bacc_hbm,
               idxr, idxc, xr, xc, outr, outc, accsh):
    cid = lax.axis_index("c")
    sid = lax.axis_index("s")
    per_s = (EP // NC) // NS
    nchunk = per_s // KE
    rows_per_s = NP // NS

    pltpu.sync_copy(z16_hbm, accsh.at[pl.ds(sid * rows_per_s, rows_per_s)])
    plsc.subcore_barrier()

    lane = lax.iota(jnp.int32, 16)
    zeros16 = jnp.zeros((16,), jnp.float32)

    @pl.loop(0, nchunk)
    def _chunk(k):
        base = cid * (EP // NC) + sid * per_s + k * KE
        pltpu.sync_copy(row_hbm.at[pl.ds(base, KE)], idxr)
        pltpu.sync_copy(col_hbm.at[pl.ds(base, KE)], idxc)
        pltpu.sync_copy(xb_hbm.at[idxr], xr)
        pltpu.sync_copy(xb_hbm.at[idxc], xc)

        @plsc.parallel_loop(0, KE, unroll=2)
        def _edge(e):
            a = xr[e, :]
            b = xc[e, :]
            d = jnp.where(lane < 3, a - b, zeros16)
            s = jnp.maximum(jnp.sum(d * d), 1e-30)
            act = jnp.sum(jnp.where(lane == 3, jnp.maximum(a, b), zeros16))
            sv = jnp.full((16,), s, jnp.float32)
            ri = jnp.int32(0x5F3759DF) - (plsc.bitcast(sv, jnp.int32) >> 1)
            r = plsc.bitcast(ri, jnp.float32)
            r = r * (1.5 - 0.5 * sv * r * r)
            r = r * (1.5 - 0.5 * sv * r * r)
            r = r * (1.5 - 0.5 * sv * r * r)
            dist = sv * r + 1e-8
            inv = 1.0 / dist
            contrib = (act * d) * ((dist - 1.5) * inv * inv)
            outr[e, :] = contrib
            outc[e, :] = -contrib

        pltpu.sync_copy(outr, accsh.at[idxr], add=True)
        pltpu.sync_copy(outc, accsh.at[idxc], add=True)

    plsc.subcore_barrier()
    pltpu.sync_copy(accsh.at[pl.ds(sid * rows_per_s, rows_per_s)],
                    bacc_hbm.at[cid, pl.ds(sid * rows_per_s, rows_per_s)])


def _bond_stage(xpadb, row_p, col_p, z16):
    f = pl.kernel(
        _bond_body,
        out_type=jax.ShapeDtypeStruct((NC, NP, 16), jnp.float32),
        mesh=_mesh,
        compiler_params=_SC_PARAMS,
        scratch_types=[
            pltpu.VMEM((KE,), jnp.int32),
            pltpu.VMEM((KE,), jnp.int32),
            pltpu.VMEM((KE, 16), jnp.float32),
            pltpu.VMEM((KE, 16), jnp.float32),
            pltpu.VMEM((KE, 16), jnp.float32),
            pltpu.VMEM((KE, 16), jnp.float32),
            pltpu.VMEM_SHARED((NP, 16), jnp.float32),
        ],
    )
    return f(xpadb, row_p, col_p, z16)


def _heads_body(h_ref, b0_ref, b1_ref, wvx_ref, bvx_ref, wa1_ref, ba1_ref,
                wa2_ref, ba2_ref, ws1_ref, bs1_ref, ws2_ref, bs2_ref,
                wcp_ref, bcp_ref, wbl_ref, bbl_ref, vx_o, va_o, vsa_o):
    h = h_ref[...]
    vxr = jnp.dot(h, wvx_ref[...],
                  preferred_element_type=jnp.float32) + bvx_ref[...][0]
    cf = jnp.dot(h, wcp_ref[...],
                 preferred_element_type=jnp.float32) + bcp_ref[...][0]
    bwl = jnp.dot(cf, wbl_ref[...],
                  preferred_element_type=jnp.float32) + bbl_ref[...][0]
    bw = jax.nn.sigmoid(bwl[:, 0:1])
    corr = b0_ref[0] + b1_ref[0]
    vx_o[...] = vxr - 0.1 * bw * corr
    ha = jnp.maximum(jnp.dot(h, wa1_ref[...],
                             preferred_element_type=jnp.float32)
                     + ba1_ref[...][0], 0.0)
    va_o[...] = jnp.dot(ha, wa2_ref[...],
                        preferred_element_type=jnp.float32) + ba2_ref[...][0]
    hs = jnp.maximum(jnp.dot(h, ws1_ref[...],
                             preferred_element_type=jnp.float32)
                     + bs1_ref[...][0], 0.0)
    vs = jnp.dot(hs, ws2_ref[...],
                 preferred_element_type=jnp.float32) + bs2_ref[...][0]
    lane = lax.broadcasted_iota(jnp.int32, vs.shape, 1)
    vsa_o[...] = vs + (lane == NBT).astype(jnp.float32)


def _heads(h, bacc, wvx16, bvx16, wa1, ba1, wa2_32, ba2_32, ws1, bs1,
           ws2_64, bs2_64, wcp, bcp, wbl16, bbl16):
    bn = 512
    return pl.pallas_call(
        _heads_body,
        grid=(NP // bn,),
        in_specs=[
            pl.BlockSpec((bn, HID), lambda i: (i, 0)),
            pl.BlockSpec((1, bn, 16), lambda i: (0, i, 0)),
            pl.BlockSpec((1, bn, 16), lambda i: (1, i, 0)),
            pl.BlockSpec((HID, 16), lambda i: (0, 0)),
            pl.BlockSpec((1, 16), lambda i: (0, 0)),
            pl.BlockSpec((HID, HID), lambda i: (0, 0)),
            pl.BlockSpec((1, HID), lambda i: (0, 0)),
            pl.BlockSpec((HID, 32), lambda i: (0, 0)),
            pl.BlockSpec((1, 32), lambda i: (0, 0)),
            pl.BlockSpec((HID, HID), lambda i: (0, 0)),
            pl.BlockSpec((1, HID), lambda i: (0, 0)),
            pl.BlockSpec((HID, 64), lambda i: (0, 0)),
            pl.BlockSpec((1, 64), lambda i: (0, 0)),
            pl.BlockSpec((HID, HID // 2), lambda i: (0, 0)),
            pl.BlockSpec((1, HID // 2), lambda i: (0, 0)),
            pl.BlockSpec((HID // 2, 16), lambda i: (0, 0)),
            pl.BlockSpec((1, 16), lambda i: (0, 0)),
        ],
        out_specs=[
            pl.BlockSpec((bn, 16), lambda i: (i, 0)),
            pl.BlockSpec((bn, 32), lambda i: (i, 0)),
            pl.BlockSpec((bn, 64), lambda i: (i, 0)),
        ],
        out_shape=[
            jax.ShapeDtypeStruct((NP, 16), jnp.float32),
            jax.ShapeDtypeStruct((NP, 32), jnp.float32),
            jax.ShapeDtypeStruct((NP, 64), jnp.float32),
        ],
    )(h, bacc, bacc, wvx16, bvx16, wa1, ba1, wa2_32, ba2_32, ws1, bs1,
      ws2_64, bs2_64, wcp, bcp, wbl16, bbl16)


def _vs_body(vsa_hbm, bid_hbm, z64_hbm, acc_hbm, idxb, buf, accsh):
    cid = lax.axis_index("c")
    sid = lax.axis_index("s")
    per_s = (NP // NC) // NS
    nchunk = per_s // 64
    rows_per_s = NBP // NS

    pltpu.sync_copy(z64_hbm, accsh.at[pl.ds(sid * rows_per_s, rows_per_s)])
    plsc.subcore_barrier()

    @pl.loop(0, nchunk)
    def _chunk(k):
        base = cid * (NP // NC) + sid * per_s + k * 64
        pltpu.sync_copy(bid_hbm.at[pl.ds(base, 64)], idxb)
        pltpu.sync_copy(vsa_hbm.at[pl.ds(base, 64)], buf)
        pltpu.sync_copy(buf, accsh.at[idxb], add=True)

    plsc.subcore_barrier()
    pltpu.sync_copy(accsh.at[pl.ds(sid * rows_per_s, rows_per_s)],
                    acc_hbm.at[cid, pl.ds(sid * rows_per_s, rows_per_s)])


def _vs_scatter(vsa, bid_p, z64):
    f = pl.kernel(
        _vs_body,
        out_type=jax.ShapeDtypeStruct((NC, NBP, 64), jnp.float32),
        mesh=_mesh,
        compiler_params=_SC_PARAMS,
        scratch_types=[
            pltpu.VMEM((64,), jnp.int32),
            pltpu.VMEM((64, 64), jnp.float32),
            pltpu.VMEM_SHARED((NBP, 64), jnp.float32),
        ],
    )
    return f(vsa, bid_p, z64)


def _vsf_body(a0_ref, a1_ref, o_ref):
    s = a0_ref[0] + a1_ref[0]
    cnt = jnp.maximum(s[:, NBT:NBT + 1], 1.0)
    o_ref[...] = s / cnt


def _vs_final(acc):
    return pl.pallas_call(
        _vsf_body,
        grid=(1,),
        in_specs=[
            pl.BlockSpec((1, NBP, 64), lambda i: (0, 0, 0)),
            pl.BlockSpec((1, NBP, 64), lambda i: (1, 0, 0)),
        ],
        out_specs=pl.BlockSpec((NBP, 64), lambda i: (0, 0)),
        out_shape=jax.ShapeDtypeStruct((NBP, 64), jnp.float32),
    )(acc, acc)


def kernel(x_t, t, atom_embed, block_embed, edge_embed, W_e2h, b_e2h, W_tp,
           b_tp, Wms1, Wmd1, Wme1, bm1, Wua1, Wub1, bu1, Wc1, Wms2, Wmd2,
           Wme2, bm2, Wua2, Wub2, bu2, Wc2, W_vx, b_vx, Wa1, ba1, Wa2, ba2,
           Ws1, bs1, Ws2, bs2, W_cp, b_cp, W_bl, b_bl, a_t, s_t, block_ids,
           batch_ids, edges, edge_type, generate_mask):
    i32 = jnp.int32
    at_p = jnp.pad(a_t.astype(i32), (0, NP - N))
    bid_p = jnp.pad(block_ids.astype(i32), (0, NP - N),
                    constant_values=NBP - 1)
    s_t2 = jnp.pad(s_t.astype(i32), (0, NBP - NB))[:, None]
    batch2 = jnp.pad(batch_ids.astype(i32), (0, NBP - NB))[:, None]
    t2 = jnp.pad(t, (0, 16 - B))[:, None]
    row_p = jnp.pad(edges[0].astype(i32), (0, EP - E), constant_values=NP - 1)
    col_p = jnp.pad(edges[1].astype(i32), (0, EP - E), constant_values=NP - 1)
    ety_p = jnp.pad(edge_type.astype(i32), (0, EP - E))
    idx3 = jnp.stack([row_p, col_p, ety_p])
    xpad0 = jnp.pad(x_t, ((0, NP - N), (0, 13)))
    gmf = generate_mask.astype(jnp.float32)[:, None]
    gmtab = jnp.pad(gmf, ((0, NBP - NB), (3, 12)))
    z128 = jnp.zeros((NP // NS, HALF), jnp.float32)
    z16 = jnp.zeros((NP // NS, 16), jnp.float32)
    z64 = jnp.zeros((NBP // NS, 64), jnp.float32)
    wvx16 = jnp.pad(W_vx, ((0, 0), (0, 13)))
    bvx16 = jnp.pad(b_vx, (0, 13))[None]
    wa2_32 = jnp.pad(Wa2, ((0, 0), (0, 2)))
    ba2_32 = jnp.pad(ba2, (0, 2))[None]
    ws2_64 = jnp.pad(Ws2, ((0, 0), (0, 64 - NBT)))
    bs2_64 = jnp.pad(bs2, (0, 64 - NBT))[None]
    wbl16 = jnp.pad(W_bl, ((0, 0), (0, 15)))
    bbl16 = jnp.pad(b_bl, (0, 15))[None]
    wub1p = Wub1
    wub2p = Wub2

    ae2, bet, et1, et2 = _tables(atom_embed, W_e2h, block_embed, s_t2,
                                 batch2, t2, b_e2h[None], b_tp[None], W_tp,
                                 edge_embed, Wme1, bm1[None], Wme2, bm2[None])
    et1_16 = et1.reshape(8, 2, HALF).transpose(1, 0, 2).reshape(16, HALF)
    et2_16 = et2.reshape(8, 2, HALF).transpose(1, 0, 2).reshape(16, HALF)

    h0, xpadb = _enc(ae2, bet, at_p, bid_p, xpad0, gmtab)

    bacc = _bond_stage(xpadb, row_p, col_p, z16)

    hs, hd = _premul(h0, Wms1, Wmd1, et1_16)
    agg1 = _edge_stage(hs.reshape(NC * NP, HALF),
                       hd.reshape(NC * 3 * NP, HALF), idx3, z128)
    h1 = _hupdate(h0, agg1, Wua1, wub1p, bu1[None])

    hs2, hd2 = _premul(h1, Wms2, Wmd2, et2_16)
    agg2 = _edge_stage(hs2.reshape(NC * NP, HALF),
                       hd2.reshape(NC * 3 * NP, HALF), idx3, z128)
    h2 = _hupdate(h1, agg2, Wua2, wub2p, bu2[None])

    vx16, va32, vsa = _heads(h2, bacc, wvx16, bvx16, Wa1, ba1[None], wa2_32,
                             ba2_32, Ws1, bs1[None], ws2_64, bs2_64,
                             W_cp, b_cp[None], wbl16, bbl16)
    vs_acc = _vs_scatter(vsa, bid_p, z64)
    vs64 = _vs_final(vs_acc)

    v_x = vx16[:N, :3]
    v_a = va32[:N, :NAT]
    v_s = vs64[:NB, :NBT]
    return (v_x, v_a, v_s)
